# baseline (speedup 1.0000x reference)
import functools

import jax
import jax.numpy as jnp
from jax import lax
from jax.experimental import pallas as pl
from jax.experimental.pallas import tpu as pltpu
from jax.experimental.pallas import tpu_sc as plsc

_N = 10000
_E = 320000
_D = 128
_NPAD = 10240
_NC = 2
_NS = 16
_NW = _NC * _NS
_EPW = _E // _NW
_C = 125
_NCHUNK = _EPW // _C
_RPS = _NPAD // _NS
_NBUF = 2
_RB = 80


def _build_segsum():
    mesh = plsc.VectorSubcoreMesh(
        core_axis_name="c", subcore_axis_name="s", num_cores=_NC)

    out_type = [
        jax.ShapeDtypeStruct((_NC, _NPAD, _D), jnp.float32),
        jax.ShapeDtypeStruct((_NC, _NPAD), jnp.float32),
    ]

    scratch_types = [
        pltpu.VMEM((2, _NBUF, 2, _C), jnp.int32),
        pltpu.VMEM((_NBUF, _C, _D), jnp.float32),
        pltpu.VMEM((128,), jnp.float32),
        pltpu.VMEM((16, _D), jnp.float32),
        pltpu.VMEM_SHARED((_NPAD, _D), jnp.float32),
        pltpu.VMEM_SHARED((_NPAD,), jnp.float32),
    ]
    scratch_types += [pltpu.SemaphoreType.DMA] * (2 * _NBUF + 3)

    _NG = _NCHUNK // _NBUF

    @functools.partial(pl.kernel, mesh=mesh, out_type=out_type,
                       scratch_types=scratch_types)
    def segsum(x_hbm, ei_hbm, agg_out, deg_out,
               idxv, rows, ones_v, zrows, acc_sh, deg_sh, *sems):
        sg, ss = sems[:_NBUF], sems[_NBUF:2 * _NBUF]
        si = sems[2 * _NBUF:2 * _NBUF + 2]
        sz = sems[2 * _NBUF + 2]

        cid = lax.axis_index("c")
        sid = lax.axis_index("s")
        wid = sid * _NC + cid

        zero16 = jnp.zeros((16,), jnp.float32)
        for r in range(16):
            for cb in range(_D // 16):
                zrows[r, pl.ds(cb * 16, 16)] = zero16
        one16 = jnp.ones((16,), jnp.float32)
        for cb in range(128 // 16):
            ones_v[pl.ds(cb * 16, 16)] = one16

        def fire_idx(g, p):
            for b in range(_NBUF):
                j = g * _NBUF + b
                pltpu.async_copy(ei_hbm.at[0, wid, j], idxv.at[p, b, 0], si[p])
                pltpu.async_copy(ei_hbm.at[1, wid, j], idxv.at[p, b, 1], si[p])

        def wait_idx(g, p):
            for b in range(_NBUF):
                j = g * _NBUF + b
                pltpu.make_async_copy(
                    ei_hbm.at[0, wid, j], idxv.at[p, b, 0], si[p]).wait()
                pltpu.make_async_copy(
                    ei_hbm.at[1, wid, j], idxv.at[p, b, 1], si[p]).wait()

        fire_idx(0, 0)
        fire_idx(1, 1)

        r0 = sid * _RPS
        zeroers = []
        for i in range(_RPS // 16):
            zeroers.append(pltpu.async_copy(
                zrows, acc_sh.at[pl.ds(r0 + i * 16, 16)], sz))
        for i in range(_RPS // _D):
            zeroers.append(pltpu.async_copy(
                zrows.at[0], deg_sh.at[pl.ds(r0 + i * _D, _D)], sz))
        for h in zeroers:
            h.wait()
        plsc.subcore_barrier()

        def ebody(k2, carry):
            for p in range(2):
                g = 2 * k2 + p
                wait_idx(g, p)
                gathers = []
                for b in range(_NBUF):
                    gathers.append(pltpu.async_copy(
                        x_hbm.at[idxv.at[p, b, 0]], rows.at[b], sg[b]))
                scatters = []
                for b in range(_NBUF):
                    gathers[b].wait()
                    scatters.append(pltpu.async_copy(
                        rows.at[b], acc_sh.at[idxv.at[p, b, 1]],
                        ss[b], add=True))
                    scatters.append(pltpu.async_copy(
                        ones_v.at[pl.ds(0, _C)], deg_sh.at[idxv.at[p, b, 1]],
                        ss[b], add=True))
                for h in scatters:
                    h.wait()

                @pl.when(g + 2 < _NG)
                def _():
                    fire_idx(g + 2, p)
            return carry

        lax.fori_loop(0, _NG // 2, ebody, None)
        plsc.subcore_barrier()

        pltpu.sync_copy(acc_sh.at[pl.ds(r0, _RPS)],
                        agg_out.at[cid, pl.ds(r0, _RPS)])
        pltpu.sync_copy(deg_sh.at[pl.ds(r0, _RPS)],
                        deg_out.at[cid, pl.ds(r0, _RPS)])

    return segsum


_CACHE = {}


def _segsum():
    if "k" not in _CACHE:
        _CACHE["k"] = _build_segsum()
    return _CACHE["k"]


def _sage_tc(aggp, degp, xin, WlT, blv, WrT):
    G = _N // _RB
    goff = _NPAD // _RB

    def body(agg_a, agg_b, deg_a, deg_b, x_ref, wl_ref, bl_ref, wr_ref, o_ref):
        agg = agg_a[...] + agg_b[...]
        deg = jnp.maximum(deg_a[...] + deg_b[...], 1.0)
        mean = agg / deg
        o_ref[...] = (
            jnp.dot(mean, wl_ref[...], preferred_element_type=jnp.float32)
            + bl_ref[...]
            + jnp.dot(x_ref[...], wr_ref[...], preferred_element_type=jnp.float32)
        )

    flat_agg = aggp.reshape(_NC * _NPAD, _D)
    flat_deg = degp.reshape(_NC * _NPAD, 1)
    return pl.pallas_call(
        body,
        grid=(G,),
        in_specs=[
            pl.BlockSpec((_RB, _D), lambda g: (g, 0)),
            pl.BlockSpec((_RB, _D), lambda g: (g + goff, 0)),
            pl.BlockSpec((_RB, 1), lambda g: (g, 0)),
            pl.BlockSpec((_RB, 1), lambda g: (g + goff, 0)),
            pl.BlockSpec((_RB, _D), lambda g: (g, 0)),
            pl.BlockSpec((_D, _D), lambda g: (0, 0)),
            pl.BlockSpec((1, _D), lambda g: (0, 0)),
            pl.BlockSpec((_D, _D), lambda g: (0, 0)),
        ],
        out_specs=pl.BlockSpec((_RB, _D), lambda g: (g, 0)),
        out_shape=jax.ShapeDtypeStruct((_N, _D), jnp.float32),
    )(flat_agg, flat_agg, flat_deg, flat_deg, xin, WlT, blv, WrT)


def kernel(x, edge_index, W1l, b1l, W1r, W2l, b2l, W2r):
    ei = edge_index.reshape(2, _NW, _NCHUNK, _C)
    segsum = _segsum()
    agg1, deg = segsum(x, ei)
    h = _sage_tc(agg1, deg, x, W1l.T, b1l.reshape(1, _D), W1r.T)
    agg2, _ = segsum(h, ei)
    return _sage_tc(agg2, deg, h, W2l.T, b2l.reshape(1, _D), W2r.T)


# speedup vs baseline: 1.0986x; 1.0986x over previous
import functools

import jax
import jax.numpy as jnp
from jax import lax
from jax.experimental import pallas as pl
from jax.experimental.pallas import tpu as pltpu
from jax.experimental.pallas import tpu_sc as plsc

_N = 10000
_E = 320000
_D = 128
_NPAD = 10240
_NC = 2
_NS = 16
_NW = _NC * _NS
_EPW = _E // _NW
_C = 125
_NCHUNK = _EPW // _C
_RPS = _NPAD // _NS
_NBUF = 2
_RB = 128


def _build_segsum():
    mesh = plsc.VectorSubcoreMesh(
        core_axis_name="c", subcore_axis_name="s", num_cores=_NC)

    out_type = [
        jax.ShapeDtypeStruct((_NC, _NPAD, _D), jnp.float32),
        jax.ShapeDtypeStruct((_NC, _NPAD), jnp.float32),
    ]

    scratch_types = [
        pltpu.VMEM((2, _NBUF, 2, _C), jnp.int32),
        pltpu.VMEM((_NBUF, _C, _D), jnp.float32),
        pltpu.VMEM((128,), jnp.float32),
        pltpu.VMEM((16, _D), jnp.float32),
        pltpu.VMEM_SHARED((_NPAD, _D), jnp.float32),
        pltpu.VMEM_SHARED((_NPAD,), jnp.float32),
    ]
    scratch_types += [pltpu.SemaphoreType.DMA] * (2 * _NBUF + 3)

    _NG = _NCHUNK // _NBUF

    @functools.partial(pl.kernel, mesh=mesh, out_type=out_type,
                       scratch_types=scratch_types)
    def segsum(x_hbm, ei_hbm, agg_out, deg_out,
               idxv, rows, ones_v, zrows, acc_sh, deg_sh, *sems):
        sg, ss = sems[:_NBUF], sems[_NBUF:2 * _NBUF]
        si = sems[2 * _NBUF:2 * _NBUF + 2]
        sz = sems[2 * _NBUF + 2]

        cid = lax.axis_index("c")
        sid = lax.axis_index("s")
        wid = sid * _NC + cid

        zero16 = jnp.zeros((16,), jnp.float32)
        for r in range(16):
            for cb in range(_D // 16):
                zrows[r, pl.ds(cb * 16, 16)] = zero16
        one16 = jnp.ones((16,), jnp.float32)
        for cb in range(128 // 16):
            ones_v[pl.ds(cb * 16, 16)] = one16

        def fire_idx(g, p):
            for b in range(_NBUF):
                j = g * _NBUF + b
                pltpu.async_copy(ei_hbm.at[0, wid, j], idxv.at[p, b, 0], si[p])
                pltpu.async_copy(ei_hbm.at[1, wid, j], idxv.at[p, b, 1], si[p])

        def wait_idx(g, p):
            for b in range(_NBUF):
                j = g * _NBUF + b
                pltpu.make_async_copy(
                    ei_hbm.at[0, wid, j], idxv.at[p, b, 0], si[p]).wait()
                pltpu.make_async_copy(
                    ei_hbm.at[1, wid, j], idxv.at[p, b, 1], si[p]).wait()

        fire_idx(0, 0)
        fire_idx(1, 1)

        r0 = sid * _RPS
        zeroers = []
        for i in range(_RPS // 16):
            zeroers.append(pltpu.async_copy(
                zrows, acc_sh.at[pl.ds(r0 + i * 16, 16)], sz))
        for i in range(_RPS // _D):
            zeroers.append(pltpu.async_copy(
                zrows.at[0], deg_sh.at[pl.ds(r0 + i * _D, _D)], sz))
        for h in zeroers:
            h.wait()
        plsc.subcore_barrier()

        def ebody(k2, carry):
            for p in range(2):
                g = 2 * k2 + p
                wait_idx(g, p)
                gathers = []
                for b in range(_NBUF):
                    gathers.append(pltpu.async_copy(
                        x_hbm.at[idxv.at[p, b, 0]], rows.at[b], sg[b]))
                scatters = []
                for b in range(_NBUF):
                    gathers[b].wait()
                    scatters.append(pltpu.async_copy(
                        rows.at[b], acc_sh.at[idxv.at[p, b, 1]],
                        ss[b], add=True))
                    scatters.append(pltpu.async_copy(
                        ones_v.at[pl.ds(0, _C)], deg_sh.at[idxv.at[p, b, 1]],
                        ss[b], add=True))
                for h in scatters:
                    h.wait()

                @pl.when(g + 2 < _NG)
                def _():
                    fire_idx(g + 2, p)
            return carry

        lax.fori_loop(0, _NG // 2, ebody, None)
        plsc.subcore_barrier()

        pltpu.sync_copy(acc_sh.at[pl.ds(r0, _RPS)],
                        agg_out.at[cid, pl.ds(r0, _RPS)])
        pltpu.sync_copy(deg_sh.at[pl.ds(r0, _RPS)],
                        deg_out.at[cid, pl.ds(r0, _RPS)])

    return segsum


_CACHE = {}


def _segsum():
    if "k" not in _CACHE:
        _CACHE["k"] = _build_segsum()
    return _CACHE["k"]


def _sage_tc(aggp, degp, xin, WlT, blv, WrT):
    G = _NPAD // _RB
    goff = _NPAD // _RB

    def body(agg_a, agg_b, deg_a, deg_b, x_ref, wl_ref, bl_ref, wr_ref, o_ref):
        agg = agg_a[...] + agg_b[...]
        deg = jnp.maximum(deg_a[...] + deg_b[...], 1.0)
        mean = agg / deg
        o_ref[...] = (
            jnp.dot(mean, wl_ref[...], preferred_element_type=jnp.float32)
            + bl_ref[...]
            + jnp.dot(x_ref[...], wr_ref[...], preferred_element_type=jnp.float32)
        )

    flat_agg = aggp.reshape(_NC * _NPAD, _D)
    flat_deg = degp.reshape(_NC * _NPAD, 1)
    return pl.pallas_call(
        body,
        grid=(G,),
        in_specs=[
            pl.BlockSpec((_RB, _D), lambda g: (g, 0)),
            pl.BlockSpec((_RB, _D), lambda g: (g + goff, 0)),
            pl.BlockSpec((_RB, 1), lambda g: (g, 0)),
            pl.BlockSpec((_RB, 1), lambda g: (g + goff, 0)),
            pl.BlockSpec((_RB, _D), lambda g: (g, 0)),
            pl.BlockSpec((_D, _D), lambda g: (0, 0)),
            pl.BlockSpec((1, _D), lambda g: (0, 0)),
            pl.BlockSpec((_D, _D), lambda g: (0, 0)),
        ],
        out_specs=pl.BlockSpec((_RB, _D), lambda g: (g, 0)),
        out_shape=jax.ShapeDtypeStruct((_NPAD, _D), jnp.float32),
    )(flat_agg, flat_agg, flat_deg, flat_deg, xin, WlT, blv, WrT)


def kernel(x, edge_index, W1l, b1l, W1r, W2l, b2l, W2r):
    ei = edge_index.reshape(2, _NW, _NCHUNK, _C)
    x_pad = jnp.pad(x, ((0, _NPAD - _N), (0, 0)))
    segsum = _segsum()
    agg1, deg = segsum(x_pad, ei)
    h = _sage_tc(agg1, deg, x_pad, W1l.T, b1l.reshape(1, _D), W1r.T)
    agg2, _ = segsum(h, ei)
    out = _sage_tc(agg2, deg, h, W2l.T, b2l.reshape(1, _D), W2r.T)
    return out[:_N]
